# baseline (speedup 1.0000x reference)
import jax
import jax.numpy as jnp
from jax.experimental import pallas as pl
from jax.experimental.pallas import tpu as pltpu

_DIM = 192
_HEADS = 4
_CH = _DIM // _HEADS
_CROP = 8
_N = _CROP * _CROP
_H = 224
_HW = _H * _H
_NW = (_H // _CROP) ** 2
_PAIRS = _NW * _HEADS

_KS_SP = (_N // 2, (_N * 2) // 3, (_N * 3) // 4, (_N * 4) // 5)
_KS_CH = (_CH // 2, (_CH * 2) // 3, (_CH * 3) // 4, (_CH * 4) // 5)

_EPS = 1e-12


def _bf(v):
    return v.astype(jnp.bfloat16)


def _multi_topk_combine(a, ks, cs):
    r_, l_ = a.shape
    lane = jax.lax.broadcasted_iota(jnp.int32, (r_, l_), 1)
    one = jnp.float32(1.0)
    zero = jnp.float32(0.0)
    rank = jnp.zeros((r_, l_), jnp.float32)
    for s in range(1, l_):
        rolled = jnp.concatenate([a[:, s:], a[:, :s]], axis=1)
        gt = rolled > a
        tie = jnp.logical_and(rolled == a, lane >= (l_ - s))
        rank = rank + jnp.where(jnp.logical_or(gt, tie), one, zero)
    m = jnp.max(a, axis=1, keepdims=True)
    e = jnp.exp(a - m)
    coef = jnp.zeros((r_, l_), jnp.float32)
    for k, c in zip(ks, cs):
        ind = jnp.where(rank < k, one, zero)
        d = jnp.sum(e * ind, axis=1, keepdims=True)
        coef = coef + ind * (c / d)
    return e * coef


def _combine_packed(a, ks, cs):
    r_, l_ = a.shape
    half = l_ // 2
    lane = jax.lax.broadcasted_iota(jnp.int32, (r_, l_), 1)
    lmod = jnp.bitwise_and(lane, half - 1)
    is_left = (lane < half).astype(jnp.float32)
    one = jnp.float32(1.0)
    zero = jnp.float32(0.0)
    rank = jnp.zeros((r_, l_), jnp.float32)
    for s in range(1, half):
        rolled = jnp.concatenate(
            [a[:, s:half], a[:, :s], a[:, half + s:], a[:, half:half + s]],
            axis=1)
        gef = jnp.where(rolled >= a, one, zero)
        gtf = jnp.where(rolled > a, one, zero)
        msk = lmod >= (half - s)
        rank = rank + jnp.where(msk, gef, gtf)
    e = jnp.exp(a)
    coef = jnp.zeros((r_, l_), jnp.float32)
    for k, c in zip(ks, cs):
        ind = jnp.where(rank < k, one, zero)
        ei = e * ind
        dt = jnp.sum(ei, axis=1, keepdims=True)
        dl = jnp.sum(ei * is_left, axis=1, keepdims=True)
        dr = dt - dl
        coef = coef + ind * jnp.where(lane < half, c / dl, c / dr)
    return e * coef


def _l2n(m):
    n = jnp.sqrt(jnp.sum(m * m, axis=1, keepdims=True))
    return m / jnp.maximum(n, _EPS)


_PB1 = 3584
_J1 = 512


def _k1_body(w_ref, x_ref, o_ref):
    wv = _bf(w_ref[...])
    for j in range(_PB1 // _J1):
        sl = pl.ds(j * _J1, _J1)
        o_ref[:, sl] = jax.lax.dot_general(
            wv, _bf(x_ref[:, sl]), (((1,), (0,)), ((), ())),
            preferred_element_type=jnp.float32).astype(jnp.bfloat16)


def _conv1x1(w, x, out_rows):
    cols = x.shape[1]
    return pl.pallas_call(
        _k1_body,
        grid=(cols // _PB1,),
        in_specs=[
            pl.BlockSpec((w.shape[0], w.shape[1]), lambda i: (0, 0)),
            pl.BlockSpec((x.shape[0], _PB1), lambda i: (0, i)),
        ],
        out_specs=pl.BlockSpec((out_rows, _PB1), lambda i: (0, i)),
        out_shape=jax.ShapeDtypeStruct((out_rows, cols), jnp.bfloat16),
    )(w, x)


_CB2 = 16
_RB2 = 16


def _k2_body(xp_ref, w_ref, o_ref, sq_ref):
    sq = jnp.zeros((_CB2, 1, 1), jnp.float32)
    for rb in range(_H // _RB2):
        acc = jnp.zeros((_CB2, _RB2, _H), jnp.float32)
        for di in range(3):
            for dj in range(3):
                tap = xp_ref[:, rb * _RB2 + di: rb * _RB2 + di + _RB2,
                             dj: dj + _H].astype(jnp.float32)
                wt = w_ref[:, 3 * di + dj: 3 * di + dj + 1]
                acc = acc + tap * wt[:, :, None]
        o_ref[:, rb * _RB2: (rb + 1) * _RB2, :] = acc
        sq = sq + jnp.sum(acc * acc, axis=(1, 2), keepdims=True)
    sq_ref[...] = sq


def _dwconv3x3(xp, w9):
    c = xp.shape[0]
    return pl.pallas_call(
        _k2_body,
        grid=(c // _CB2,),
        in_specs=[
            pl.BlockSpec((_CB2, _H + 2, _H + 2), lambda i: (i, 0, 0)),
            pl.BlockSpec((_CB2, 9), lambda i: (i, 0)),
        ],
        out_specs=[
            pl.BlockSpec((_CB2, _H, _H), lambda i: (i, 0, 0)),
            pl.BlockSpec((_CB2, 1, 1), lambda i: (i, 0, 0)),
        ],
        out_shape=[
            jax.ShapeDtypeStruct((c, _H, _H), jnp.float32),
            jax.ShapeDtypeStruct((c, 1, 1), jnp.float32),
        ],
    )(xp, w9)


_WB = 16


def _k3_body(par_ref, tmat_ref, q_ref, k_ref, v_ref, o_ref):
    qn = _bf(_l2n(q_ref[...]))
    kn = _bf(_l2n(k_ref[...]))
    vv = v_ref[...]
    rowsets = []
    for t in range(_WB // 2):
        pair = []
        for p in (2 * t, 2 * t + 1):
            pair.append(jax.lax.dot_general(
                qn[p * _N:(p + 1) * _N, :], kn[p * _N:(p + 1) * _N, :],
                (((1,), (1,)), ((), ())), preferred_element_type=jnp.float32))
        rowsets.append(jnp.concatenate(pair, axis=1))
    att = jnp.concatenate(rowsets, axis=0) * tmat_ref[...]
    cs = (par_ref[4], par_ref[5], par_ref[6], par_ref[7])
    pmat = _bf(_combine_packed(att, _KS_SP, cs))
    z = jnp.zeros((_N, _CH), jnp.bfloat16)
    for t in range(_WB // 2):
        v0 = vv[(2 * t) * _N:(2 * t + 1) * _N, :]
        v1 = vv[(2 * t + 1) * _N:(2 * t + 2) * _N, :]
        vd = jnp.concatenate(
            [jnp.concatenate([v0, z], axis=1),
             jnp.concatenate([z, v1], axis=1)], axis=0)
        o_ref[t * _N:(t + 1) * _N, :] = jax.lax.dot_general(
            pmat[t * _N:(t + 1) * _N, :], vd,
            (((1,), (0,)), ((), ())), preferred_element_type=jnp.float32)


def _spatial_attn(params, tmat, qs, ks, vs):
    rows = qs.shape[0]
    blk = _WB * _N
    orows = rows // 2
    oblk = (_WB // 2) * _N
    return pl.pallas_call(
        _k3_body,
        grid=(rows // blk,),
        in_specs=[
            pl.BlockSpec(memory_space=pltpu.SMEM),
            pl.BlockSpec(((_WB // 2) * _N, 2 * _N), lambda i: (0, 0)),
            pl.BlockSpec((blk, _CH), lambda i: (i, 0)),
            pl.BlockSpec((blk, _CH), lambda i: (i, 0)),
            pl.BlockSpec((blk, _CH), lambda i: (i, 0)),
        ],
        out_specs=pl.BlockSpec((oblk, 2 * _CH), lambda i: (i, 0)),
        out_shape=jax.ShapeDtypeStruct((orows, 2 * _CH), jnp.float32),
    )(params, tmat, qs, ks, vs)


_PB4 = 3584


def _k4a_body(q_ref, k_ref, sq_ref, sk_ref):
    i = pl.program_id(0)

    @pl.when(i == 0)
    def _init():
        sq_ref[...] = jnp.zeros_like(sq_ref)
        sk_ref[...] = jnp.zeros_like(sk_ref)

    qv = q_ref[...]
    kv = k_ref[...]
    sq_ref[...] += jnp.sum(qv * qv, axis=1, keepdims=True)
    sk_ref[...] += jnp.sum(kv * kv, axis=1, keepdims=True)


def _rownorms(qc, kc):
    return pl.pallas_call(
        _k4a_body,
        grid=(_HW // _PB4,),
        in_specs=[
            pl.BlockSpec((_DIM, _PB4), lambda i: (0, i)),
            pl.BlockSpec((_DIM, _PB4), lambda i: (0, i)),
        ],
        out_specs=[
            pl.BlockSpec((_DIM, 1), lambda i: (0, 0)),
            pl.BlockSpec((_DIM, 1), lambda i: (0, 0)),
        ],
        out_shape=[
            jax.ShapeDtypeStruct((_DIM, 1), jnp.float32),
            jax.ShapeDtypeStruct((_DIM, 1), jnp.float32),
        ],
    )(qc, kc)


def _k4b_body(par_ref, sq_ref, sk_ref, q_ref, k_ref, o_ref, g_ref):
    i = pl.program_id(0)
    nsteps = pl.num_programs(0)

    @pl.when(i == 0)
    def _init():
        g_ref[...] = jnp.zeros_like(g_ref)

    nq = jnp.maximum(jnp.sqrt(sq_ref[...]), _EPS)
    nk = jnp.maximum(jnp.sqrt(sk_ref[...]), _EPS)
    qn = _bf(q_ref[...] / nq)
    kn = _bf(k_ref[...] / nk)
    for h in range(_HEADS):
        sl = pl.ds(h * _CH, _CH)
        g_ref[sl, :] += jax.lax.dot_general(
            qn[h * _CH:(h + 1) * _CH, :], kn[h * _CH:(h + 1) * _CH, :],
            (((1,), (1,)), ((), ())), preferred_element_type=jnp.float32)

    @pl.when(i == nsteps - 1)
    def _fin():
        parts = []
        for h in range(_HEADS):
            parts.append(g_ref[h * _CH:(h + 1) * _CH, :] * par_ref[h])
        att = jnp.concatenate(parts, axis=0)
        att = jnp.concatenate(
            [att, jnp.full((_DIM, _N - _CH), -jnp.inf, jnp.float32)], axis=1)
        cs = (par_ref[4], par_ref[5], par_ref[6], par_ref[7])
        pmat = _multi_topk_combine(att, _KS_CH, cs)
        o_ref[...] = pmat[:, :_CH]


def _channel_pmat(params, sq, sk, qc, kc):
    return pl.pallas_call(
        _k4b_body,
        grid=(_HW // _PB4,),
        in_specs=[
            pl.BlockSpec(memory_space=pltpu.SMEM),
            pl.BlockSpec((_DIM, 1), lambda i: (0, 0)),
            pl.BlockSpec((_DIM, 1), lambda i: (0, 0)),
            pl.BlockSpec((_DIM, _PB4), lambda i: (0, i)),
            pl.BlockSpec((_DIM, _PB4), lambda i: (0, i)),
        ],
        out_specs=pl.BlockSpec((_DIM, _CH), lambda i: (0, 0)),
        out_shape=jax.ShapeDtypeStruct((_DIM, _CH), jnp.float32),
        scratch_shapes=[
            pltpu.VMEM((_DIM, _CH), jnp.float32),
        ],
    )(params, sq, sk, qc, kc)


_PB5 = 3584
_J5 = 512


def _k5_body(pc_ref, v_ref, sp_ref, w_ref, o_ref):
    wv = _bf(w_ref[...])
    pc = _bf(pc_ref[...])
    for j in range(_PB5 // _J5):
        sl = pl.ds(j * _J5, _J5)
        vb = v_ref[:, sl]
        parts = []
        for h in range(_HEADS):
            u_h = jax.lax.dot_general(
                pc[h * _CH:(h + 1) * _CH, :],
                vb[h * _CH:(h + 1) * _CH, :],
                (((1,), (0,)), ((), ())), preferred_element_type=jnp.float32)
            parts.append(u_h + sp_ref[h * _CH:(h + 1) * _CH, sl])
        u = jnp.concatenate(parts, axis=0)
        o_ref[:, sl] = jax.lax.dot_general(
            wv, _bf(u), (((1,), (0,)), ((), ())),
            preferred_element_type=jnp.float32)


def _combine_proj(pc, vc, sp, wp):
    return pl.pallas_call(
        _k5_body,
        grid=(_HW // _PB5,),
        in_specs=[
            pl.BlockSpec((_DIM, _CH), lambda i: (0, 0)),
            pl.BlockSpec((_DIM, _PB5), lambda i: (0, i)),
            pl.BlockSpec((_DIM, _PB5), lambda i: (0, i)),
            pl.BlockSpec((_DIM, _DIM), lambda i: (0, 0)),
        ],
        out_specs=pl.BlockSpec((_DIM, _PB5), lambda i: (0, i)),
        out_shape=jax.ShapeDtypeStruct((_DIM, _HW), jnp.float32),
    )(pc, vc, sp, wp)


def kernel(x, qkv_w, dw_w, proj_w, temperature, attn1, attn2, attn3, attn4):
    b, c, h, w = x.shape

    params = jnp.concatenate([
        temperature.reshape(_HEADS).astype(jnp.float32),
        attn1.reshape(1), attn2.reshape(1),
        attn3.reshape(1), attn4.reshape(1)]).astype(jnp.float32)

    xf = x.reshape(c, _HW)
    wqkv = qkv_w.reshape(3 * c, c)
    qkv = _conv1x1(wqkv, xf, 3 * c)

    qkv_pad = jnp.pad(qkv.reshape(3 * c, _H, _H),
                      ((0, 0), (1, 1), (1, 1)))
    w9 = dw_w.reshape(3 * c, 9)
    qkv2, sqkv = _dwconv3x3(qkv_pad, w9)
    qkv2 = qkv2.reshape(3 * c, _HW)
    sqkv = sqkv.reshape(3 * c, 1)

    q = qkv2[:c]
    k = qkv2[c:2 * c]
    v = qkv2[2 * c:]
    vbf = v.astype(jnp.bfloat16)

    def towin(m):
        r = m.reshape(1, _HEADS, _CH, _H // _CROP, _CROP, _H // _CROP, _CROP)
        p = jnp.transpose(r, (0, 1, 3, 5, 4, 6, 2))
        return p.reshape(_PAIRS * _N, _CH)

    qs = towin(q)
    ks = towin(k)
    vs = towin(vbf)

    tvals = temperature.reshape(_HEADS).astype(jnp.float32)
    left = jnp.repeat(tvals[jnp.arange(0, _WB, 2) % _HEADS], _N)[:, None]
    right = jnp.repeat(tvals[jnp.arange(1, _WB, 2) % _HEADS], _N)[:, None]
    ones_n = jnp.ones((1, _N), jnp.float32)
    tmat = jnp.concatenate([left * ones_n, right * ones_n], axis=1)

    sp_packed = _spatial_attn(params, tmat, qs, ks, vs)
    sp = jnp.transpose(
        sp_packed.reshape(_PAIRS // 2, _N, 2, _CH), (0, 2, 1, 3))

    wt = sp.reshape(_NW, _HEADS, _N, _CH)
    r = wt.reshape(1, _HEADS, _H, _H, _CH)
    sp_img = jnp.transpose(r, (0, 1, 4, 2, 3)).reshape(_DIM, _HW)

    pc = _channel_pmat(params, sqkv[:c], sqkv[c:2 * c], q, k)
    out = _combine_proj(pc, vbf, sp_img, proj_w.reshape(_DIM, _DIM))

    return out.reshape(b, c, h, w)


# speedup vs baseline: 1.0206x; 1.0206x over previous
import jax
import jax.numpy as jnp
from jax.experimental import pallas as pl
from jax.experimental.pallas import tpu as pltpu

_DIM = 192
_HEADS = 4
_CH = _DIM // _HEADS
_CROP = 8
_N = _CROP * _CROP
_H = 224
_HW = _H * _H
_NW = (_H // _CROP) ** 2
_PAIRS = _NW * _HEADS

_KS_SP = (_N // 2, (_N * 2) // 3, (_N * 3) // 4, (_N * 4) // 5)
_KS_CH = (_CH // 2, (_CH * 2) // 3, (_CH * 3) // 4, (_CH * 4) // 5)

_EPS = 1e-12


def _bf(v):
    return v.astype(jnp.bfloat16)


def _multi_topk_combine(a, ks, cs):
    r_, l_ = a.shape
    lane = jax.lax.broadcasted_iota(jnp.int32, (r_, l_), 1)
    one = jnp.float32(1.0)
    zero = jnp.float32(0.0)
    rank = jnp.zeros((r_, l_), jnp.float32)
    for s in range(1, l_):
        rolled = jnp.concatenate([a[:, s:], a[:, :s]], axis=1)
        gt = rolled > a
        tie = jnp.logical_and(rolled == a, lane >= (l_ - s))
        rank = rank + jnp.where(jnp.logical_or(gt, tie), one, zero)
    m = jnp.max(a, axis=1, keepdims=True)
    e = jnp.exp(a - m)
    coef = jnp.zeros((r_, l_), jnp.float32)
    for k, c in zip(ks, cs):
        ind = jnp.where(rank < k, one, zero)
        d = jnp.sum(e * ind, axis=1, keepdims=True)
        coef = coef + ind * (c / d)
    return e * coef


def _combine_packed(a, ks, cs):
    r_, l_ = a.shape
    half = l_ // 2
    lane = jax.lax.broadcasted_iota(jnp.int32, (r_, l_), 1)
    lmod = jnp.bitwise_and(lane, half - 1)
    is_left = (lane < half).astype(jnp.float32)
    one = jnp.float32(1.0)
    zero = jnp.float32(0.0)
    rank = jnp.zeros((r_, l_), jnp.float32)
    for s in range(1, half):
        rolled = jnp.concatenate(
            [a[:, s:half], a[:, :s], a[:, half + s:], a[:, half:half + s]],
            axis=1)
        gef = jnp.where(rolled >= a, one, zero)
        gtf = jnp.where(rolled > a, one, zero)
        msk = lmod >= (half - s)
        rank = rank + jnp.where(msk, gef, gtf)
    e = jnp.exp(a)
    coef = jnp.zeros((r_, l_), jnp.float32)
    for k, c in zip(ks, cs):
        ind = jnp.where(rank < k, one, zero)
        ei = e * ind
        dt = jnp.sum(ei, axis=1, keepdims=True)
        dl = jnp.sum(ei * is_left, axis=1, keepdims=True)
        dr = dt - dl
        coef = coef + ind * jnp.where(lane < half, c / dl, c / dr)
    return e * coef


def _l2n(m):
    n = jnp.sqrt(jnp.sum(m * m, axis=1, keepdims=True))
    return m / jnp.maximum(n, _EPS)


_PB1 = 3584
_J1 = 512


def _k1_body(w_ref, x_ref, o_ref):
    wv = _bf(w_ref[...])
    for j in range(_PB1 // _J1):
        sl = pl.ds(j * _J1, _J1)
        o_ref[:, sl] = jax.lax.dot_general(
            wv, _bf(x_ref[:, sl]), (((1,), (0,)), ((), ())),
            preferred_element_type=jnp.float32).astype(jnp.bfloat16)


def _conv1x1(w, x, out_rows):
    cols = x.shape[1]
    return pl.pallas_call(
        _k1_body,
        grid=(cols // _PB1,),
        in_specs=[
            pl.BlockSpec((w.shape[0], w.shape[1]), lambda i: (0, 0)),
            pl.BlockSpec((x.shape[0], _PB1), lambda i: (0, i)),
        ],
        out_specs=pl.BlockSpec((out_rows, _PB1), lambda i: (0, i)),
        out_shape=jax.ShapeDtypeStruct((out_rows, cols), jnp.bfloat16),
    )(w, x)


_CB2 = 16
_RB2 = 16


def _k2_body(xp_ref, w_ref, o_ref):
    for rb in range(_H // _RB2):
        acc = jnp.zeros((_CB2, _RB2, _H), jnp.float32)
        for di in range(3):
            for dj in range(3):
                tap = xp_ref[:, rb * _RB2 + di: rb * _RB2 + di + _RB2,
                             dj: dj + _H].astype(jnp.float32)
                wt = w_ref[:, 3 * di + dj: 3 * di + dj + 1]
                acc = acc + tap * wt[:, :, None]
        o_ref[:, rb * _RB2: (rb + 1) * _RB2, :] = acc


def _dwconv3x3(xp, w9):
    c = xp.shape[0]
    return pl.pallas_call(
        _k2_body,
        grid=(c // _CB2,),
        in_specs=[
            pl.BlockSpec((_CB2, _H + 2, _H + 2), lambda i: (i, 0, 0)),
            pl.BlockSpec((_CB2, 9), lambda i: (i, 0)),
        ],
        out_specs=pl.BlockSpec((_CB2, _H, _H), lambda i: (i, 0, 0)),
        out_shape=jax.ShapeDtypeStruct((c, _H, _H), jnp.float32),
    )(xp, w9)


_WB = 16


def _k3_body(par_ref, tmat_ref, q_ref, k_ref, v_ref, o_ref):
    qn = _bf(_l2n(q_ref[...]))
    kn = _bf(_l2n(k_ref[...]))
    vv = v_ref[...]
    rowsets = []
    for t in range(_WB // 2):
        pair = []
        for p in (2 * t, 2 * t + 1):
            pair.append(jax.lax.dot_general(
                qn[p * _N:(p + 1) * _N, :], kn[p * _N:(p + 1) * _N, :],
                (((1,), (1,)), ((), ())), preferred_element_type=jnp.float32))
        rowsets.append(jnp.concatenate(pair, axis=1))
    att = jnp.concatenate(rowsets, axis=0) * tmat_ref[...]
    cs = (par_ref[4], par_ref[5], par_ref[6], par_ref[7])
    pmat = _bf(_combine_packed(att, _KS_SP, cs))
    z = jnp.zeros((_N, _CH), jnp.bfloat16)
    for t in range(_WB // 2):
        v0 = vv[(2 * t) * _N:(2 * t + 1) * _N, :]
        v1 = vv[(2 * t + 1) * _N:(2 * t + 2) * _N, :]
        vd = jnp.concatenate(
            [jnp.concatenate([v0, z], axis=1),
             jnp.concatenate([z, v1], axis=1)], axis=0)
        o_ref[t * _N:(t + 1) * _N, :] = jax.lax.dot_general(
            pmat[t * _N:(t + 1) * _N, :], vd,
            (((1,), (0,)), ((), ())), preferred_element_type=jnp.float32)


def _spatial_attn(params, tmat, qs, ks, vs):
    rows = qs.shape[0]
    blk = _WB * _N
    orows = rows // 2
    oblk = (_WB // 2) * _N
    return pl.pallas_call(
        _k3_body,
        grid=(rows // blk,),
        in_specs=[
            pl.BlockSpec(memory_space=pltpu.SMEM),
            pl.BlockSpec(((_WB // 2) * _N, 2 * _N), lambda i: (0, 0)),
            pl.BlockSpec((blk, _CH), lambda i: (i, 0)),
            pl.BlockSpec((blk, _CH), lambda i: (i, 0)),
            pl.BlockSpec((blk, _CH), lambda i: (i, 0)),
        ],
        out_specs=pl.BlockSpec((oblk, 2 * _CH), lambda i: (i, 0)),
        out_shape=jax.ShapeDtypeStruct((orows, 2 * _CH), jnp.float32),
    )(params, tmat, qs, ks, vs)


_PB4 = 3584


def _k4a_body(q_ref, k_ref, sq_ref, sk_ref):
    i = pl.program_id(0)

    @pl.when(i == 0)
    def _init():
        sq_ref[...] = jnp.zeros_like(sq_ref)
        sk_ref[...] = jnp.zeros_like(sk_ref)

    qv = q_ref[...]
    kv = k_ref[...]
    sq_ref[...] += jnp.sum(qv * qv, axis=1, keepdims=True)
    sk_ref[...] += jnp.sum(kv * kv, axis=1, keepdims=True)


def _rownorms(qc, kc):
    return pl.pallas_call(
        _k4a_body,
        grid=(_HW // _PB4,),
        in_specs=[
            pl.BlockSpec((_DIM, _PB4), lambda i: (0, i)),
            pl.BlockSpec((_DIM, _PB4), lambda i: (0, i)),
        ],
        out_specs=[
            pl.BlockSpec((_DIM, 1), lambda i: (0, 0)),
            pl.BlockSpec((_DIM, 1), lambda i: (0, 0)),
        ],
        out_shape=[
            jax.ShapeDtypeStruct((_DIM, 1), jnp.float32),
            jax.ShapeDtypeStruct((_DIM, 1), jnp.float32),
        ],
    )(qc, kc)


def _k4b_body(par_ref, sq_ref, sk_ref, q_ref, k_ref, o_ref, g_ref):
    i = pl.program_id(0)
    nsteps = pl.num_programs(0)

    @pl.when(i == 0)
    def _init():
        g_ref[...] = jnp.zeros_like(g_ref)

    nq = jnp.maximum(jnp.sqrt(sq_ref[...]), _EPS)
    nk = jnp.maximum(jnp.sqrt(sk_ref[...]), _EPS)
    qn = _bf(q_ref[...] / nq)
    kn = _bf(k_ref[...] / nk)
    for h in range(_HEADS):
        sl = pl.ds(h * _CH, _CH)
        g_ref[sl, :] += jax.lax.dot_general(
            qn[h * _CH:(h + 1) * _CH, :], kn[h * _CH:(h + 1) * _CH, :],
            (((1,), (1,)), ((), ())), preferred_element_type=jnp.float32)

    @pl.when(i == nsteps - 1)
    def _fin():
        parts = []
        for h in range(_HEADS):
            parts.append(g_ref[h * _CH:(h + 1) * _CH, :] * par_ref[h])
        att = jnp.concatenate(parts, axis=0)
        att = jnp.concatenate(
            [att, jnp.full((_DIM, _N - _CH), -jnp.inf, jnp.float32)], axis=1)
        cs = (par_ref[4], par_ref[5], par_ref[6], par_ref[7])
        pmat = _multi_topk_combine(att, _KS_CH, cs)
        o_ref[...] = pmat[:, :_CH]


def _channel_pmat(params, sq, sk, qc, kc):
    return pl.pallas_call(
        _k4b_body,
        grid=(_HW // _PB4,),
        in_specs=[
            pl.BlockSpec(memory_space=pltpu.SMEM),
            pl.BlockSpec((_DIM, 1), lambda i: (0, 0)),
            pl.BlockSpec((_DIM, 1), lambda i: (0, 0)),
            pl.BlockSpec((_DIM, _PB4), lambda i: (0, i)),
            pl.BlockSpec((_DIM, _PB4), lambda i: (0, i)),
        ],
        out_specs=pl.BlockSpec((_DIM, _CH), lambda i: (0, 0)),
        out_shape=jax.ShapeDtypeStruct((_DIM, _CH), jnp.float32),
        scratch_shapes=[
            pltpu.VMEM((_DIM, _CH), jnp.float32),
        ],
    )(params, sq, sk, qc, kc)


_PB5 = 3584
_J5 = 512


def _k5_body(pc_ref, v_ref, sp_ref, w_ref, o_ref):
    wv = _bf(w_ref[...])
    pc = _bf(pc_ref[...])
    for j in range(_PB5 // _J5):
        sl = pl.ds(j * _J5, _J5)
        vb = _bf(v_ref[:, sl])
        parts = []
        for h in range(_HEADS):
            u_h = jax.lax.dot_general(
                pc[h * _CH:(h + 1) * _CH, :],
                vb[h * _CH:(h + 1) * _CH, :],
                (((1,), (0,)), ((), ())), preferred_element_type=jnp.float32)
            parts.append(u_h + sp_ref[h * _CH:(h + 1) * _CH, sl])
        u = jnp.concatenate(parts, axis=0)
        o_ref[:, sl] = jax.lax.dot_general(
            wv, _bf(u), (((1,), (0,)), ((), ())),
            preferred_element_type=jnp.float32)


def _combine_proj(pc, vc, sp, wp):
    return pl.pallas_call(
        _k5_body,
        grid=(_HW // _PB5,),
        in_specs=[
            pl.BlockSpec((_DIM, _CH), lambda i: (0, 0)),
            pl.BlockSpec((_DIM, _PB5), lambda i: (0, i)),
            pl.BlockSpec((_DIM, _PB5), lambda i: (0, i)),
            pl.BlockSpec((_DIM, _DIM), lambda i: (0, 0)),
        ],
        out_specs=pl.BlockSpec((_DIM, _PB5), lambda i: (0, i)),
        out_shape=jax.ShapeDtypeStruct((_DIM, _HW), jnp.float32),
    )(pc, vc, sp, wp)


def kernel(x, qkv_w, dw_w, proj_w, temperature, attn1, attn2, attn3, attn4):
    b, c, h, w = x.shape

    params = jnp.concatenate([
        temperature.reshape(_HEADS).astype(jnp.float32),
        attn1.reshape(1), attn2.reshape(1),
        attn3.reshape(1), attn4.reshape(1)]).astype(jnp.float32)

    xf = x.reshape(c, _HW)
    wqkv = qkv_w.reshape(3 * c, c)
    qkv = _conv1x1(wqkv, xf, 3 * c)

    qkv_pad = jnp.pad(qkv.reshape(3 * c, _H, _H),
                      ((0, 0), (1, 1), (1, 1)))
    w9 = dw_w.reshape(3 * c, 9)
    qkv2 = _dwconv3x3(qkv_pad, w9).reshape(3 * c, _HW)

    q = qkv2[:c]
    k = qkv2[c:2 * c]
    v = qkv2[2 * c:]

    def towin(m):
        r = m.reshape(1, _HEADS, _CH, _H // _CROP, _CROP, _H // _CROP, _CROP)
        p = jnp.transpose(r, (0, 1, 3, 5, 4, 6, 2))
        return p.reshape(_PAIRS * _N, _CH)

    qs = towin(q)
    ks = towin(k)
    vs = towin(v.astype(jnp.bfloat16))

    tvals = temperature.reshape(_HEADS).astype(jnp.float32)
    left = jnp.repeat(tvals[jnp.arange(0, _WB, 2) % _HEADS], _N)[:, None]
    right = jnp.repeat(tvals[jnp.arange(1, _WB, 2) % _HEADS], _N)[:, None]
    ones_n = jnp.ones((1, _N), jnp.float32)
    tmat = jnp.concatenate([left * ones_n, right * ones_n], axis=1)

    sp_packed = _spatial_attn(params, tmat, qs, ks, vs)
    sp = jnp.transpose(
        sp_packed.reshape(_PAIRS // 2, _N, 2, _CH), (0, 2, 1, 3))

    wt = sp.reshape(_NW, _HEADS, _N, _CH)
    r = wt.reshape(1, _HEADS, _H, _H, _CH)
    sp_img = jnp.transpose(r, (0, 1, 4, 2, 3)).reshape(_DIM, _HW)

    sq, sk = _rownorms(q, k)
    pc = _channel_pmat(params, sq, sk, q, k)
    out = _combine_proj(pc, v, sp_img, proj_w.reshape(_DIM, _DIM))

    return out.reshape(b, c, h, w)
